# baseline (speedup 1.0000x reference)
import functools

import jax
import jax.numpy as jnp
from jax import lax
from jax.experimental import pallas as pl
from jax.experimental.pallas import tpu as pltpu
from jax.experimental.pallas import tpu_sc as plsc


def _sc_gather(table, idx, T, D):
    info = plsc.get_sparse_core_info()
    NW = info.num_cores * info.num_subcores
    b_per_w = T // NW

    mesh = plsc.VectorSubcoreMesh(core_axis_name="c", subcore_axis_name="s")

    @functools.partial(
        pl.kernel,
        mesh=mesh,
        out_type=jax.ShapeDtypeStruct((T, D), jnp.float32),
        scratch_types=[
            pltpu.VMEM((b_per_w,), jnp.int32),
            pltpu.VMEM((b_per_w, D), jnp.float32),
            pltpu.SemaphoreType.DMA,
        ],
        compiler_params=pltpu.CompilerParams(use_tc_tiling_on_sc=False),
    )
    def gather_kernel(table_hbm, idx_hbm, out_hbm, idx_v, rows_v, sem):
        wid = lax.axis_index("s") * info.num_cores + lax.axis_index("c")
        base = wid * b_per_w
        pltpu.sync_copy(idx_hbm.at[pl.ds(base, b_per_w)], idx_v)
        pltpu.async_copy(table_hbm.at[idx_v], rows_v, sem).wait()
        pltpu.sync_copy(rows_v, out_hbm.at[pl.ds(base, b_per_w)])

    return gather_kernel(table, idx)


def _tc_project_t(x, W, b, vt=1600):
    T, D = x.shape
    V = W.shape[0]
    nv = pl.cdiv(V, vt)

    def mm(w_ref, x_ref, b_ref, o_ref):
        o_ref[...] = lax.dot_general(
            w_ref[...], x_ref[...],
            (((1,), (1,)), ((), ())),
            preferred_element_type=jnp.float32,
        ) + b_ref[...]

    return pl.pallas_call(
        mm,
        grid=(nv,),
        in_specs=[
            pl.BlockSpec((vt, D), lambda j: (j, 0)),
            pl.BlockSpec((T, D), lambda j: (0, 0)),
            pl.BlockSpec((vt, 1), lambda j: (j, 0)),
        ],
        out_specs=pl.BlockSpec((vt, T), lambda j: (j, 0)),
        out_shape=jax.ShapeDtypeStruct((V, T), jnp.float32),
        compiler_params=pltpu.CompilerParams(
            dimension_semantics=("parallel",),
        ),
    )(W.astype(jnp.bfloat16), x.astype(jnp.bfloat16), b.reshape(V, 1))


def kernel(input_ids, embed_table, W_proj, b_proj):
    B, T = input_ids.shape
    V, D = embed_table.shape
    ids = input_ids.reshape(T).astype(jnp.int32)
    x = _sc_gather(embed_table, ids, T, D)
    logits_t = _tc_project_t(x, W_proj, b_proj)
    return logits_t.T.reshape(B, T, V)
